# Initial kernel scaffold; baseline (speedup 1.0000x reference)
import jax
import jax.numpy as jnp
from jax.experimental import pallas as pl


def kernel(x, id_table, W1, b1):
    raise NotImplementedError("write your pallas kernel here")


# speedup vs baseline: 2.6713x; 2.6713x over previous
import math

import jax
import jax.numpy as jnp
from jax import lax
from jax.experimental import pallas as pl

_EMBED_DIM = 768
_ID_DIM = 64
_HALF = _EMBED_DIM // 2
_QUARTER = _HALF // 2
_TABLE_ROWS = 1000
_TABLE_PAD = 1024


def _body(x0_ref, t_ref, tab_ref, w0_ref, w1_ref, b_ref, out_ref):
    x0 = x0_ref[:, :]
    t = t_ref[:, :]
    u = x0 * w0_ref[:, :] + t * w1_ref[:, :] + b_ref[:, :]

    scale = -math.log(10000.0) / (_QUARTER - 1)
    col = lax.broadcasted_iota(jnp.int32, (1, _QUARTER), 1).astype(jnp.float32)
    freqs = jnp.exp(col * scale)
    emb = t * freqs
    v_sin = jnp.sin(emb)
    v_cos = jnp.cos(emb)

    idx = jnp.clip(x0.astype(jnp.int32), 0, _TABLE_ROWS - 1)
    r = x0.shape[0]
    row_iota = lax.broadcasted_iota(jnp.int32, (r, _TABLE_PAD), 1)
    onehot = (row_iota == idx).astype(jnp.float32)
    i_emb = jnp.dot(onehot, tab_ref[:, :],
                    preferred_element_type=jnp.float32)

    out_ref[:, :] = jnp.concatenate([i_emb, u, v_sin, v_cos], axis=1)


def kernel(x, id_table, W1, b1):
    batch, _, seq = x.shape
    n = batch * seq
    block_r = 1024
    grid = n // block_r

    x0 = x[:, 0, :].reshape(n, 1)
    t = x[:, 1, :].reshape(n, 1)
    tab = jnp.concatenate(
        [id_table, jnp.zeros((_TABLE_PAD - _TABLE_ROWS, _ID_DIM),
                             dtype=id_table.dtype)], axis=0)
    w0 = W1[0].reshape(1, _HALF)
    w1 = W1[1].reshape(1, _HALF)
    b = b1.reshape(1, _HALF)

    out = pl.pallas_call(
        _body,
        grid=(grid,),
        in_specs=[
            pl.BlockSpec((block_r, 1), lambda i: (i, 0)),
            pl.BlockSpec((block_r, 1), lambda i: (i, 0)),
            pl.BlockSpec((_TABLE_PAD, _ID_DIM), lambda i: (0, 0)),
            pl.BlockSpec((1, _HALF), lambda i: (0, 0)),
            pl.BlockSpec((1, _HALF), lambda i: (0, 0)),
            pl.BlockSpec((1, _HALF), lambda i: (0, 0)),
        ],
        out_specs=pl.BlockSpec((block_r, _EMBED_DIM + _ID_DIM),
                               lambda i: (i, 0)),
        out_shape=jax.ShapeDtypeStruct((n, _EMBED_DIM + _ID_DIM),
                                       jnp.float32),
    )(x0, t, tab, w0, w1, b)

    return out.reshape(batch, seq, _EMBED_DIM + _ID_DIM)
